# Initial kernel scaffold; baseline (speedup 1.0000x reference)
import jax
import jax.numpy as jnp
from jax.experimental import pallas as pl


def kernel(x, edge_index, W1, b1, W2, b2):
    raise NotImplementedError("write your pallas kernel here")


# speedup vs baseline: 34.1951x; 34.1951x over previous
import functools

import jax
import jax.numpy as jnp
from jax import lax
from jax.experimental import pallas as pl
from jax.experimental.pallas import tpu as pltpu
from jax.experimental.pallas import tpu_sc as plsc

N_TILES = 32
N_SUB = 16
K = 128
LANES = 16


def _mesh():
    return plsc.VectorSubcoreMesh(core_axis_name="c", subcore_axis_name="s")


def _make_deg_kernel(npad, ch):
    sl = npad // N_SUB

    @functools.partial(
        pl.kernel,
        out_type=jax.ShapeDtypeStruct((2, npad), jnp.float32),
        mesh=_mesh(),
        compiler_params=pltpu.CompilerParams(needs_layout_passes=False),
        scratch_types=[
            pltpu.VMEM_SHARED((npad,), jnp.float32),
            pltpu.VMEM((ch, K), jnp.int32),
            pltpu.VMEM((K,), jnp.float32),
        ],
    )
    def deg_kernel(dst_hbm, zvec_hbm, out_hbm, deg_sp, dst_v, ones_v):
        c = lax.axis_index("c")
        s = lax.axis_index("s")
        w = c * N_SUB + s
        pltpu.sync_copy(zvec_hbm.at[pl.ds(s * sl, sl)], deg_sp.at[pl.ds(s * sl, sl)])
        pltpu.sync_copy(dst_hbm.at[w], dst_v)
        for k in range(K // LANES):
            ones_v[pl.ds(k * LANES, LANES)] = jnp.full((LANES,), 1.0, jnp.float32)
        plsc.subcore_barrier()

        def body(i, carry):
            pltpu.sync_copy(ones_v, deg_sp.at[dst_v.at[i]], add=True)
            return carry

        lax.fori_loop(0, ch, body, 0)
        plsc.subcore_barrier()
        pltpu.sync_copy(deg_sp.at[pl.ds(s * sl, sl)], out_hbm.at[c, pl.ds(s * sl, sl)])

    return deg_kernel


def _make_row_scatter_kernel(npad, d, ch):
    sl = npad // N_SUB

    @functools.partial(
        pl.kernel,
        out_type=jax.ShapeDtypeStruct((2, npad, d), jnp.float32),
        mesh=_mesh(),
        compiler_params=pltpu.CompilerParams(needs_layout_passes=False),
        scratch_types=[
            pltpu.VMEM_SHARED((npad, d), jnp.float32),
            pltpu.VMEM((ch, K), jnp.int32),
            pltpu.VMEM((ch, K), jnp.int32),
            pltpu.VMEM((K, d), jnp.float32),
            pltpu.VMEM((K, d), jnp.float32),
            pltpu.SemaphoreType.DMA,
            pltpu.SemaphoreType.DMA,
        ],
    )
    def scat_kernel(g_hbm, src_hbm, dst_hbm, zrows_hbm, out_hbm,
                    acc_sp, src_v, dst_v, rows_a, rows_b, sem_a, sem_b):
        c = lax.axis_index("c")
        s = lax.axis_index("s")
        w = c * N_SUB + s
        pltpu.sync_copy(zrows_hbm.at[pl.ds(s * sl, sl)], acc_sp.at[pl.ds(s * sl, sl)])
        pltpu.sync_copy(src_hbm.at[w], src_v)
        pltpu.sync_copy(dst_hbm.at[w], dst_v)
        plsc.subcore_barrier()

        def body2(i, carry):
            pltpu.async_copy(g_hbm.at[src_v.at[i]], rows_a, sem_a).wait()
            pltpu.sync_copy(rows_a, acc_sp.at[dst_v.at[i]], add=True)
            return carry

        lax.fori_loop(0, ch, body2, 0)
        plsc.subcore_barrier()
        pltpu.sync_copy(acc_sp.at[pl.ds(s * sl, sl)], out_hbm.at[c, pl.ds(s * sl, sl)])

    return scat_kernel


def _make_scalar_kernel(npad, ch):
    sl = npad // N_SUB

    @functools.partial(
        pl.kernel,
        out_type=jax.ShapeDtypeStruct((2, npad), jnp.float32),
        mesh=_mesh(),
        compiler_params=pltpu.CompilerParams(needs_layout_passes=False),
        scratch_types=[
            pltpu.VMEM_SHARED((npad,), jnp.float32),
            pltpu.VMEM((ch, K), jnp.int32),
            pltpu.VMEM((ch, K), jnp.int32),
            pltpu.VMEM((npad // 128, 128), jnp.float32),
            pltpu.VMEM((K,), jnp.float32),
            pltpu.VMEM((sl,), jnp.float32),
            pltpu.VMEM((sl,), jnp.float32),
            pltpu.VMEM((LANES,), jnp.float32),
        ],
    )
    def l2agg_kernel(z_hbm, dinv_hbm, b2_hbm, src_hbm, dst_hbm, zvec_hbm, out_hbm,
                     acc_sp, src_v, dst_v, z_v, upd_v, acc_v, dinv_v, b2_v):
        c = lax.axis_index("c")
        s = lax.axis_index("s")
        pltpu.sync_copy(zvec_hbm.at[pl.ds(s * sl, sl)], acc_sp.at[pl.ds(s * sl, sl)])
        pltpu.sync_copy(src_hbm.at[s], src_v)
        pltpu.sync_copy(dst_hbm.at[s], dst_v)
        pltpu.sync_copy(z_hbm, z_v)
        pltpu.sync_copy(dinv_hbm.at[pl.ds(s * sl, sl)], dinv_v)
        pltpu.sync_copy(b2_hbm, b2_v)
        plsc.subcore_barrier()

        def body(i, carry):
            for k in range(K // LANES):
                s16 = src_v[i, pl.ds(k * LANES, LANES)]
                r16 = lax.shift_right_logical(s16, 7)
                c16 = lax.bitwise_and(s16, 127)
                upd_v[pl.ds(k * LANES, LANES)] = plsc.load_gather(z_v, [r16, c16])
            pltpu.sync_copy(upd_v, acc_sp.at[dst_v.at[i]], add=True)
            return carry

        lax.fori_loop(0, ch, body, 0)
        plsc.subcore_barrier()
        pltpu.sync_copy(acc_sp.at[pl.ds(s * sl, sl)], acc_v)
        b2 = b2_v[...]
        rows_per_sub = sl // 128
        for t in range(sl // LANES):
            a16 = acc_v[pl.ds(t * LANES, LANES)]
            z16 = z_v[s * rows_per_sub + t // 8, pl.ds((t % 8) * LANES, LANES)]
            d16 = dinv_v[pl.ds(t * LANES, LANES)]
            acc_v[pl.ds(t * LANES, LANES)] = b2 + d16 * (a16 + z16)
        pltpu.sync_copy(acc_v, out_hbm.at[c, pl.ds(s * sl, sl)])

    return l2agg_kernel


def _dinv_body(deg_ref, out_ref):
    out_ref[...] = lax.rsqrt(deg_ref[0] + deg_ref[1] + 1.0)


def _g_body(x_ref, w1_ref, dinv_ref, out_ref):
    h = jnp.dot(x_ref[...], w1_ref[...], preferred_element_type=jnp.float32)
    out_ref[...] = dinv_ref[...] * h


def _l4_body(acc_ref, g_ref, dinv_ref, w2_ref, b1_ref, z_ref):
    d = dinv_ref[...]
    h1 = jnp.maximum(d * (acc_ref[0] + acc_ref[1] + g_ref[...]) + b1_ref[...], 0.0)
    z_ref[...] = d * jnp.dot(h1, w2_ref[...], preferred_element_type=jnp.float32)


def kernel(x, edge_index, W1, b1, W2, b2):
    n, d_in = x.shape
    d_hid = W1.shape[1]
    ei = edge_index.astype(jnp.int32)
    src, dst = ei[0], ei[1]
    e = src.shape[0]

    npad = ((n + 16 * 40 - 1) // (16 * 40)) * (16 * 40)
    ch = (e + N_TILES * K - 1) // (N_TILES * K)
    e_pad = N_TILES * ch * K
    n_extra = e_pad - e
    pad_idx = jnp.arange(n_extra, dtype=jnp.int32)
    src_p = jnp.concatenate([src, pad_idx % n])
    dst_p = jnp.concatenate([dst, n + pad_idx % (npad - n)])
    src32 = src_p.reshape(N_TILES, ch, K)
    dst32 = dst_p.reshape(N_TILES, ch, K)
    ch5 = e_pad // (N_SUB * K)
    src16 = src_p.reshape(N_SUB, ch5, K)
    dst16 = dst_p.reshape(N_SUB, ch5, K)

    zvec = jnp.zeros((npad,), jnp.float32)
    zrows = jnp.zeros((npad, d_hid), jnp.float32)
    x_pad = jnp.pad(x, ((0, npad - n), (0, 0)))

    deg = _make_deg_kernel(npad, ch)(dst32, zvec)

    dinv80 = pl.pallas_call(
        _dinv_body,
        out_shape=jax.ShapeDtypeStruct((npad // 128, 128), jnp.float32),
    )(deg.reshape(2, npad // 128, 128))
    dinv_col = dinv80.reshape(npad, 1)
    dinv_flat = dinv80.reshape(npad)

    rb = 512
    grid = (npad // rb,)
    g = pl.pallas_call(
        _g_body,
        grid=grid,
        in_specs=[
            pl.BlockSpec((rb, d_in), lambda i: (i, 0)),
            pl.BlockSpec((d_in, d_hid), lambda i: (0, 0)),
            pl.BlockSpec((rb, 1), lambda i: (i, 0)),
        ],
        out_specs=pl.BlockSpec((rb, d_hid), lambda i: (i, 0)),
        out_shape=jax.ShapeDtypeStruct((npad, d_hid), jnp.float32),
    )(x_pad, W1, dinv_col)

    acc = _make_row_scatter_kernel(npad, d_hid, ch)(g, src32, dst32, zrows)

    z2d = pl.pallas_call(
        _l4_body,
        grid=grid,
        in_specs=[
            pl.BlockSpec((2, rb, d_hid), lambda i: (0, i, 0)),
            pl.BlockSpec((rb, d_hid), lambda i: (i, 0)),
            pl.BlockSpec((rb, 1), lambda i: (i, 0)),
            pl.BlockSpec((d_hid, 1), lambda i: (0, 0)),
            pl.BlockSpec((1, d_hid), lambda i: (0, 0)),
        ],
        out_specs=pl.BlockSpec((rb, 1), lambda i: (i, 0)),
        out_shape=jax.ShapeDtypeStruct((npad, 1), jnp.float32),
    )(acc, g, dinv_col, W2, b1.reshape(1, d_hid))
    z_flat = z2d.reshape(npad)

    b2_b = jnp.broadcast_to(b2, (LANES,)).astype(jnp.float32)
    out5 = _make_scalar_kernel(npad, ch5)(
        z_flat.reshape(npad // 128, 128), dinv_flat, b2_b, src16, dst16, zvec)

    return out5[0, :n]
